# Initial kernel scaffold; baseline (speedup 1.0000x reference)
import jax
import jax.numpy as jnp
from jax.experimental import pallas as pl


def kernel(values, alpha, temp, gamma_hs, U, V, ln_w, ln_b):
    raise NotImplementedError("write your pallas kernel here")


# speedup vs baseline: 16.9257x; 16.9257x over previous
import functools
from math import sqrt

import jax
import jax.numpy as jnp
from jax.experimental import pallas as pl

_N_ITERS = 32


def _softplus(x):
    return jnp.maximum(x, 0.0) + jnp.log1p(jnp.exp(-jnp.abs(x)))


def _masked_softmax_topk(stack, k, n_iters):
    m = jax.lax.bitcast_convert_type(stack, jnp.int32)
    m = m ^ ((m >> 31) & jnp.int32(0x7FFFFFFF))

    rows = stack.shape[0]
    lo0 = jnp.full((rows, 1), jnp.iinfo(jnp.int32).min, dtype=jnp.int32)
    hi0 = jnp.full((rows, 1), jnp.iinfo(jnp.int32).max, dtype=jnp.int32)

    def body(_, carry):
        lo, hi = carry
        mid = (lo >> 1) + (hi >> 1) + (lo & hi & 1)
        cnt = jnp.sum((m >= mid).astype(jnp.int32), axis=-1, keepdims=True)
        pred = cnt >= k
        return jnp.where(pred, mid, lo), jnp.where(pred, hi, mid)

    lo, _ = jax.lax.fori_loop(0, n_iters, body, (lo0, hi0), unroll=True)

    mask = m >= lo
    xm = jnp.where(mask, stack, -jnp.inf)
    rmax = jnp.max(xm, axis=-1, keepdims=True)
    e = jnp.exp(xm - rmax)
    return e / jnp.sum(e, axis=-1, keepdims=True)


def _gating_kernel(alpha_ref, values_ref, temp_ref, u_ref, v_ref, lnw_ref,
                   lnb_ref, out_ref, *, k, sblk, f, d, n_iters):
    scale = 1.0 / sqrt(f)
    h = pl.program_id(0)
    lane = jax.lax.broadcasted_iota(jnp.int32, temp_ref.shape, 1)
    t_h = jnp.sum(jnp.where(lane == h, temp_ref[...], 0.0))
    gain = _softplus(t_h)

    ones_d = jnp.ones((1, d), dtype=jnp.float32)
    dim_d = (((1,), (1,)), ((), ()))

    def score_row(vals):
        vsq = vals * vals
        energy = jax.lax.dot_general(
            ones_d, vsq, dim_d, precision=jax.lax.Precision.HIGHEST) / d
        rms = jnp.maximum(jnp.sqrt(jnp.mean(energy)), 1e-6)
        s = energy / rms * gain
        mu = jnp.mean(s)
        var = jnp.mean((s - mu) ** 2)
        return (s - mu) / jnp.sqrt(var + 1e-5) * lnw_ref[0, :] + lnb_ref[0, :]

    vals0 = values_ref[0, 0]
    vals1 = values_ref[1, 0]
    sc0 = score_row(vals0)
    sc1 = score_row(vals1)

    bil = jax.lax.dot_general(
        u_ref[0], v_ref[0], (((1,), (0,)), ((), ())),
        precision=jax.lax.Precision.HIGHEST)

    al = alpha_ref[0] * scale
    stack = jnp.concatenate([al, bil + sc0, bil + sc1], axis=0)

    p = _masked_softmax_topk(stack, k, n_iters)
    attn_a = p[:sblk]
    mix0 = p[sblk:2 * sblk] + attn_a
    mix1 = p[2 * sblk:] + attn_a

    out_ref[0, 0] = jnp.dot(mix0, vals0, precision=jax.lax.Precision.HIGHEST)
    out_ref[1, 0] = jnp.dot(mix1, vals1, precision=jax.lax.Precision.HIGHEST)


def kernel(values, alpha, temp, gamma_hs, U, V, ln_w, ln_b):
    del gamma_hs
    H, S, F = alpha.shape
    B = values.shape[0]
    D = values.shape[3]
    k = max(1, int(0.1 * F))
    sblk = 256 if S % 256 == 0 else S

    grid = (H, S // sblk)
    body = functools.partial(_gating_kernel, k=k, sblk=sblk, f=F, d=D,
                             n_iters=_N_ITERS)
    values_t = jnp.transpose(values, (0, 2, 1, 3))
    out = pl.pallas_call(
        body,
        grid=grid,
        in_specs=[
            pl.BlockSpec((1, sblk, F), lambda h, sb: (h, sb, 0)),
            pl.BlockSpec((B, 1, F, D), lambda h, sb: (0, h, 0, 0)),
            pl.BlockSpec((1, H), lambda h, sb: (0, 0)),
            pl.BlockSpec((1, sblk, U.shape[2]), lambda h, sb: (h, sb, 0)),
            pl.BlockSpec((1, V.shape[1], F), lambda h, sb: (h, 0, 0)),
            pl.BlockSpec((1, F), lambda h, sb: (0, 0)),
            pl.BlockSpec((1, F), lambda h, sb: (0, 0)),
        ],
        out_specs=pl.BlockSpec((B, 1, sblk, D), lambda h, sb: (0, h, sb, 0)),
        out_shape=jax.ShapeDtypeStruct((B, H, S, D), jnp.float32),
    )(alpha, values_t, temp.reshape(1, H), U, V,
      ln_w.reshape(1, F), ln_b.reshape(1, F))
    return jnp.transpose(out, (0, 2, 1, 3))


# speedup vs baseline: 25.4702x; 1.5048x over previous
import functools
from math import sqrt

import jax
import jax.numpy as jnp
from jax.experimental import pallas as pl

_N_ITERS = 24


def _softplus(x):
    return jnp.maximum(x, 0.0) + jnp.log1p(jnp.exp(-jnp.abs(x)))


def _masked_softmax_topk_t(stack, k, n_iters):
    m = jax.lax.bitcast_convert_type(stack, jnp.int32)
    m = m ^ ((m >> 31) & jnp.int32(0x7FFFFFFF))

    rows = stack.shape[1]
    lo0 = jnp.full((1, rows), jnp.iinfo(jnp.int32).min, dtype=jnp.int32)
    hi0 = jnp.full((1, rows), jnp.iinfo(jnp.int32).max, dtype=jnp.int32)
    kf = jnp.float32(k)

    def body(_, carry):
        lo, hi = carry
        mid = (lo >> 1) + (hi >> 1) + (lo & hi & 1)
        cnt = jnp.sum(jnp.where(m >= mid, 1.0, 0.0), axis=0, keepdims=True)
        pred = cnt >= kf
        return jnp.where(pred, mid, lo), jnp.where(pred, hi, mid)

    lo, _ = jax.lax.fori_loop(0, n_iters, body, (lo0, hi0), unroll=True)

    mask = m >= lo
    xm = jnp.where(mask, stack, -jnp.inf)
    cmax = jnp.max(xm, axis=0, keepdims=True)
    e = jnp.exp(xm - cmax)
    return e / jnp.sum(e, axis=0, keepdims=True)


def _gating_kernel(alpha_ref, values_ref, temp_ref, u_ref, v_ref, lnw_ref,
                   lnb_ref, out_ref, *, k, sblk, f, d, n_iters):
    scale = 1.0 / sqrt(f)
    h = pl.program_id(0)
    lane = jax.lax.broadcasted_iota(jnp.int32, temp_ref.shape, 1)
    t_h = jnp.sum(jnp.where(lane == h, temp_ref[...], 0.0))
    gain = _softplus(t_h)

    ones_d = jnp.ones((d, 1), dtype=jnp.float32)

    def score_col(vals):
        vsq = vals * vals
        energy = jnp.dot(vsq, ones_d,
                         precision=jax.lax.Precision.HIGHEST) / d
        rms = jnp.maximum(jnp.sqrt(jnp.mean(energy)), 1e-6)
        s = energy / rms * gain
        mu = jnp.mean(s)
        var = jnp.mean((s - mu) ** 2)
        return (s - mu) / jnp.sqrt(var + 1e-5) * lnw_ref[...] + lnb_ref[...]

    vals0 = values_ref[0, 0]
    vals1 = values_ref[1, 0]
    sc0 = score_col(vals0)
    sc1 = score_col(vals1)

    bil = jnp.dot(v_ref[0], u_ref[0], precision=jax.lax.Precision.HIGHEST)

    stack = jnp.concatenate(
        [alpha_ref[0] * scale, bil + sc0, bil + sc1], axis=1)

    p = _masked_softmax_topk_t(stack, k, n_iters)
    attn_a = p[:, :sblk]
    mix0 = p[:, sblk:2 * sblk] + attn_a
    mix1 = p[:, 2 * sblk:] + attn_a

    dim_ff = (((0,), (0,)), ((), ()))
    out_ref[0, 0] = jax.lax.dot_general(mix0, vals0, dim_ff,
                                        precision=jax.lax.Precision.HIGHEST)
    out_ref[1, 0] = jax.lax.dot_general(mix1, vals1, dim_ff,
                                        precision=jax.lax.Precision.HIGHEST)


def kernel(values, alpha, temp, gamma_hs, U, V, ln_w, ln_b):
    del gamma_hs
    H, S, F = alpha.shape
    B = values.shape[0]
    D = values.shape[3]
    R = U.shape[2]
    k = max(1, int(0.1 * F))
    sblk = 256 if S % 256 == 0 else S

    grid = (H, S // sblk)
    body = functools.partial(_gating_kernel, k=k, sblk=sblk, f=F, d=D,
                             n_iters=_N_ITERS)
    values_t = jnp.transpose(values, (0, 2, 1, 3))
    alpha_t = jnp.transpose(alpha, (0, 2, 1))
    u_t = jnp.transpose(U, (0, 2, 1))
    v_t = jnp.transpose(V, (0, 2, 1))
    out = pl.pallas_call(
        body,
        grid=grid,
        in_specs=[
            pl.BlockSpec((1, F, sblk), lambda h, sb: (h, 0, sb)),
            pl.BlockSpec((B, 1, F, D), lambda h, sb: (0, h, 0, 0)),
            pl.BlockSpec((1, H), lambda h, sb: (0, 0)),
            pl.BlockSpec((1, R, sblk), lambda h, sb: (h, 0, sb)),
            pl.BlockSpec((1, F, R), lambda h, sb: (h, 0, 0)),
            pl.BlockSpec((F, 1), lambda h, sb: (0, 0)),
            pl.BlockSpec((F, 1), lambda h, sb: (0, 0)),
        ],
        out_specs=pl.BlockSpec((B, 1, sblk, D), lambda h, sb: (0, h, sb, 0)),
        out_shape=jax.ShapeDtypeStruct((B, H, S, D), jnp.float32),
    )(alpha_t, values_t, temp.reshape(1, H), u_t, v_t,
      ln_w.reshape(F, 1), ln_b.reshape(F, 1))
    return jnp.transpose(out, (0, 2, 1, 3))
